# baseline (speedup 1.0000x reference)
import functools

import jax
import jax.numpy as jnp
import numpy as np
from jax import lax
from jax.experimental import pallas as pl
from jax.experimental.pallas import tpu as pltpu
from jax.sharding import Mesh, PartitionSpec as P

E = 8
TOPK = 2
H = 2048
F = 2048
FT = 512
NF = F // FT


def _moe_body(eoff_ref, x_ref, tw_ref, te_ref, w1_ref, v1_ref, w2_ref,
              out_ref):
    e = pl.program_id(0)
    f = pl.program_id(1)

    @pl.when((e == 0) & (f == 0))
    def _init():
        out_ref[:] = jnp.zeros_like(out_ref)

    dn = (((1,), (1,)), ((), ()))
    x = x_ref[:, 0, :]
    xw = jax.lax.dot_general(x, w1_ref[:], dn,
                             preferred_element_type=jnp.float32)
    xv = jax.lax.dot_general(x, v1_ref[:], dn,
                             preferred_element_type=jnp.float32)
    inter = xw * jax.nn.sigmoid(xw) * xv

    sel = te_ref[:] == (e + eoff_ref[0])
    coef = jnp.sum(jnp.where(sel, tw_ref[:], 0.0), axis=-1)
    inter = inter * coef[:, None]

    out_ref[:, 0, :] += jnp.dot(inter, w2_ref[:],
                                preferred_element_type=jnp.float32)


def _local_moe(el, bsz, q_len, x, top_weights, top_experts, w1s, v1s, w2s):
    T = bsz * q_len
    eoff = jnp.reshape(lax.axis_index("d") * el, (1,)).astype(jnp.int32)
    wspec = pl.BlockSpec((FT, H), lambda e, f, *_: (e * NF + f, 0))
    grid_spec = pltpu.PrefetchScalarGridSpec(
        num_scalar_prefetch=1,
        grid=(el, NF),
        in_specs=[
            pl.BlockSpec((bsz, q_len, H), lambda e, f, *_: (0, 0, 0)),
            pl.BlockSpec((T, TOPK), lambda e, f, *_: (0, 0)),
            pl.BlockSpec((T, TOPK), lambda e, f, *_: (0, 0)),
            wspec,
            wspec,
            wspec,
        ],
        out_specs=pl.BlockSpec((bsz, q_len, H), lambda e, f, *_: (0, 0, 0)),
    )
    part = pl.pallas_call(
        _moe_body,
        grid_spec=grid_spec,
        out_shape=jax.ShapeDtypeStruct((bsz, q_len, H), jnp.float32),
        compiler_params=pltpu.CompilerParams(
            dimension_semantics=("arbitrary", "arbitrary"),
            vmem_limit_bytes=120 * 1024 * 1024,
        ),
    )(eoff, x, top_weights, top_experts, w1s, v1s, w2s)
    return lax.psum(part, "d")


def kernel(x, weights, top_weights, top_experts, w1, v1, w2):
    bsz, q_len, hidden = x.shape
    devs = jax.devices()
    nd = 2 if len(devs) >= 2 else 1
    el = E // nd
    mesh = Mesh(np.array(devs[:nd]), ("d",))
    fn = jax.shard_map(
        functools.partial(_local_moe, el, bsz, q_len),
        mesh=mesh,
        in_specs=(P(), P(), P(), P("d"), P("d"), P("d")),
        out_specs=P(),
        check_vma=False,
    )
    return fn(x, top_weights, top_experts, w1, v1, w2)


# speedup vs baseline: 5.5621x; 5.5621x over previous
import functools

import jax
import jax.numpy as jnp
from jax.experimental import pallas as pl
from jax.experimental.pallas import tpu as pltpu

E = 8
TOPK = 2
H = 2048
F = 2048
FT = 512
NF = F // FT


def _moe_body(x_ref, tw_ref, te_ref, w1_ref, v1_ref, w2_ref, out_ref):
    e = pl.program_id(0)
    f = pl.program_id(1)

    @pl.when((e == 0) & (f == 0))
    def _init():
        out_ref[:] = jnp.zeros_like(out_ref)

    dn = (((1,), (1,)), ((), ()))
    x = x_ref[:, 0, :]
    xw = jax.lax.dot_general(x, w1_ref[:], dn,
                             preferred_element_type=jnp.float32)
    xv = jax.lax.dot_general(x, v1_ref[:], dn,
                             preferred_element_type=jnp.float32)
    inter = xw * jax.nn.sigmoid(xw) * xv

    sel = te_ref[:] == e
    coef = jnp.sum(jnp.where(sel, tw_ref[:], 0.0), axis=-1)
    inter = inter * coef[:, None]

    out_ref[:, 0, :] += jnp.dot(inter, w2_ref[:],
                                preferred_element_type=jnp.float32)


def kernel(x, weights, top_weights, top_experts, w1, v1, w2):
    bsz, q_len, hidden = x.shape
    T = bsz * q_len

    wspec = pl.BlockSpec((FT, H), lambda e, f: (e * NF + f, 0))
    grid = (E, NF)
    out = pl.pallas_call(
        _moe_body,
        grid=grid,
        in_specs=[
            pl.BlockSpec((bsz, q_len, H), lambda e, f: (0, 0, 0)),
            pl.BlockSpec((T, TOPK), lambda e, f: (0, 0)),
            pl.BlockSpec((T, TOPK), lambda e, f: (0, 0)),
            wspec,
            wspec,
            wspec,
        ],
        out_specs=pl.BlockSpec((bsz, q_len, H), lambda e, f: (0, 0, 0)),
        out_shape=jax.ShapeDtypeStruct((bsz, q_len, H), jnp.float32),
        compiler_params=pltpu.CompilerParams(
            dimension_semantics=("arbitrary", "arbitrary"),
            vmem_limit_bytes=120 * 1024 * 1024,
        ),
    )(x, top_weights, top_experts, w1, v1, w2)
    return out
